# initial kernel scaffold (unmeasured)
import jax
import jax.numpy as jnp
from jax import lax
from jax.experimental import pallas as pl
from jax.experimental.pallas import tpu as pltpu


def kernel(
    x,
):
    def body(*refs):
        pass

    out_shape = jax.ShapeDtypeStruct(..., jnp.float32)
    return pl.pallas_call(body, out_shape=out_shape)(...)


# baseline (device time: 190564 ns/iter reference)
import numpy as np

import jax
import jax.numpy as jnp
from jax import lax
from jax.experimental import pallas as pl
from jax.experimental.pallas import tpu as pltpu

N_DEV = 4


def _cmp_ex_const(x, K, j):
    n, C = x.shape
    nb = n // (2 * j)
    a = x.reshape(nb, 2, j, C)
    lo = jnp.minimum(a[:, 0], a[:, 1])
    hi = jnp.maximum(a[:, 0], a[:, 1])
    if (nb - 1) * 2 * j < K:
        first, second = lo, hi
    else:
        shift = (K // (2 * j)).bit_length() - 1
        b = lax.broadcasted_iota(jnp.int32, (nb, 1, 1), 0)
        m = (b >> shift) % 2 == 1
        first = jnp.where(m, hi, lo)
        second = jnp.where(m, lo, hi)
    return jnp.stack([first, second], axis=1).reshape(n, C)


def _cmp_ex_dir(x, j, desc):
    n, C = x.shape
    nb = n // (2 * j)
    a = x.reshape(nb, 2, j, C)
    lo = jnp.minimum(a[:, 0], a[:, 1])
    hi = jnp.maximum(a[:, 0], a[:, 1])
    first = jnp.where(desc, hi, lo)
    second = jnp.where(desc, lo, hi)
    return jnp.stack([first, second], axis=1).reshape(n, C)


def _merge_dir(x, desc):
    j = x.shape[0] // 2
    while j >= 1:
        x = _cmp_ex_dir(x, j, desc)
        j //= 2
    return x


def kernel(x):
    m_per, n_cols = x.shape

    def body(x_ref, out_ref, work_ref, comm_ref, send_sems, recv_sems):
        my = lax.axis_index("i")
        p1 = my ^ 1
        p2 = my ^ 2

        barrier_sem = pltpu.get_barrier_semaphore()
        for nbr in (p1, p2):
            pl.semaphore_signal(
                barrier_sem, inc=1,
                device_id=(nbr,), device_id_type=pl.DeviceIdType.MESH,
            )
        pl.semaphore_wait(barrier_sem, 2)

        def exchange(step, partner, keep_max, val):
            work_ref[...] = val
            rdma = pltpu.make_async_remote_copy(
                src_ref=work_ref,
                dst_ref=comm_ref.at[step],
                send_sem=send_sems.at[step],
                recv_sem=recv_sems.at[step],
                device_id=(partner,),
                device_id_type=pl.DeviceIdType.MESH,
            )
            rdma.start()
            rdma.wait()
            r = comm_ref[step]
            return jnp.where(keep_max, jnp.maximum(val, r), jnp.minimum(val, r))

        xv = x_ref[...].astype(jnp.bfloat16)

        K = 2
        while K <= m_per // 2:
            j = K // 2
            while j >= 1:
                xv = _cmp_ex_const(xv, K, j)
                j //= 2
            K *= 2
        xv = _merge_dir(xv, my % 2 == 1)

        xv = exchange(0, p1, (my == 1) | (my == 2), xv)
        xv = _merge_dir(xv, my >= 2)

        xv = exchange(1, p2, my >= 2, xv)
        xv = exchange(2, p1, my % 2 == 1, xv)
        j = m_per // 2
        while j >= 1:
            xv = _cmp_ex_const(xv, 2 * m_per, j)
            j //= 2

        out_ref[...] = xv.astype(jnp.float32)

    return pl.pallas_call(
        body,
        out_shape=jax.ShapeDtypeStruct((m_per, n_cols), jnp.float32),
        in_specs=[pl.BlockSpec(memory_space=pltpu.VMEM)],
        out_specs=pl.BlockSpec(memory_space=pltpu.VMEM),
        scratch_shapes=[
            pltpu.VMEM((m_per, n_cols), jnp.bfloat16),
            pltpu.VMEM((3, m_per, n_cols), jnp.bfloat16),
            pltpu.SemaphoreType.DMA((3,)),
            pltpu.SemaphoreType.DMA((3,)),
        ],
        compiler_params=pltpu.CompilerParams(collective_id=0),
    )(x)


# device time: 151726 ns/iter; 1.2560x vs baseline; 1.2560x over previous
import numpy as np

import jax
import jax.numpy as jnp
from jax import lax
from jax.experimental import pallas as pl
from jax.experimental.pallas import tpu as pltpu

N_DEV = 4


_ROLL_MAX_J = 8


def _stage_roll(x, j, K, desc_traced):
    n, C = x.shape
    i = lax.broadcasted_iota(jnp.int32, (n, 1), 0)
    bitj = (i >> (j.bit_length() - 1)) % 2 == 1
    up = jnp.roll(x, -j, axis=0)
    dn = jnp.roll(x, j, axis=0)
    partner = jnp.where(bitj, dn, up)
    lo = jnp.minimum(x, partner)
    hi = jnp.maximum(x, partner)
    keep_hi = bitj
    if K is not None and K < n:
        keep_hi = keep_hi ^ ((i >> (K.bit_length() - 1)) % 2 == 1)
    if desc_traced is not None:
        keep_hi = keep_hi ^ desc_traced
    return jnp.where(keep_hi, hi, lo)


def _cmp_ex_const(x, K, j):
    n, C = x.shape
    if j < _ROLL_MAX_J:
        return _stage_roll(x, j, K, None)
    nb = n // (2 * j)
    a = x.reshape(nb, 2, j, C)
    lo = jnp.minimum(a[:, 0], a[:, 1])
    hi = jnp.maximum(a[:, 0], a[:, 1])
    if (nb - 1) * 2 * j < K:
        first, second = lo, hi
    else:
        shift = (K // (2 * j)).bit_length() - 1
        b = lax.broadcasted_iota(jnp.int32, (nb, 1, 1), 0)
        m = (b >> shift) % 2 == 1
        first = jnp.where(m, hi, lo)
        second = jnp.where(m, lo, hi)
    return jnp.stack([first, second], axis=1).reshape(n, C)


def _cmp_ex_dir(x, j, desc):
    n, C = x.shape
    if j < _ROLL_MAX_J:
        return _stage_roll(x, j, None, desc)
    nb = n // (2 * j)
    a = x.reshape(nb, 2, j, C)
    lo = jnp.minimum(a[:, 0], a[:, 1])
    hi = jnp.maximum(a[:, 0], a[:, 1])
    first = jnp.where(desc, hi, lo)
    second = jnp.where(desc, lo, hi)
    return jnp.stack([first, second], axis=1).reshape(n, C)


def _merge_dir(x, desc):
    j = x.shape[0] // 2
    while j >= 1:
        x = _cmp_ex_dir(x, j, desc)
        j //= 2
    return x


def kernel(x):
    m_per, n_cols = x.shape

    def body(x_ref, out_ref, work_ref, comm_ref, send_sems, recv_sems):
        my = lax.axis_index("i")
        p1 = my ^ 1
        p2 = my ^ 2

        barrier_sem = pltpu.get_barrier_semaphore()
        for nbr in (p1, p2):
            pl.semaphore_signal(
                barrier_sem, inc=1,
                device_id=(nbr,), device_id_type=pl.DeviceIdType.MESH,
            )
        pl.semaphore_wait(barrier_sem, 2)

        def exchange(step, partner, keep_max, val):
            work_ref[...] = val
            rdma = pltpu.make_async_remote_copy(
                src_ref=work_ref,
                dst_ref=comm_ref.at[step],
                send_sem=send_sems.at[step],
                recv_sem=recv_sems.at[step],
                device_id=(partner,),
                device_id_type=pl.DeviceIdType.MESH,
            )
            rdma.start()
            rdma.wait()
            r = comm_ref[step]
            return jnp.where(keep_max, jnp.maximum(val, r), jnp.minimum(val, r))

        xv = x_ref[...].astype(jnp.bfloat16)

        K = 2
        while K <= m_per // 2:
            j = K // 2
            while j >= 1:
                xv = _cmp_ex_const(xv, K, j)
                j //= 2
            K *= 2
        xv = _merge_dir(xv, my % 2 == 1)

        xv = exchange(0, p1, (my == 1) | (my == 2), xv)
        xv = _merge_dir(xv, my >= 2)

        xv = exchange(1, p2, my >= 2, xv)
        xv = exchange(2, p1, my % 2 == 1, xv)
        j = m_per // 2
        while j >= 1:
            xv = _cmp_ex_const(xv, 2 * m_per, j)
            j //= 2

        out_ref[...] = xv.astype(jnp.float32)

    return pl.pallas_call(
        body,
        out_shape=jax.ShapeDtypeStruct((m_per, n_cols), jnp.float32),
        in_specs=[pl.BlockSpec(memory_space=pltpu.VMEM)],
        out_specs=pl.BlockSpec(memory_space=pltpu.VMEM),
        scratch_shapes=[
            pltpu.VMEM((m_per, n_cols), jnp.bfloat16),
            pltpu.VMEM((3, m_per, n_cols), jnp.bfloat16),
            pltpu.SemaphoreType.DMA((3,)),
            pltpu.SemaphoreType.DMA((3,)),
        ],
        compiler_params=pltpu.CompilerParams(collective_id=0),
    )(x)


# device time: 147199 ns/iter; 1.2946x vs baseline; 1.0308x over previous
import numpy as np

import jax
import jax.numpy as jnp
from jax import lax
from jax.experimental import pallas as pl
from jax.experimental.pallas import tpu as pltpu

N_DEV = 4


_ROLL_MAX_J = 8


def _stage_roll(x, j, K, desc_traced):
    n, C = x.shape
    i = lax.broadcasted_iota(jnp.int32, (n, 1), 0)
    bitj = (i >> (j.bit_length() - 1)) % 2 == 1
    up = jnp.roll(x, -j, axis=0)
    dn = jnp.roll(x, j, axis=0)
    partner = jnp.where(bitj, dn, up)
    lo = jnp.minimum(x, partner)
    hi = jnp.maximum(x, partner)
    keep_hi = bitj
    if K is not None and K < n:
        keep_hi = keep_hi ^ ((i >> (K.bit_length() - 1)) % 2 == 1)
    if desc_traced is not None:
        keep_hi = keep_hi ^ desc_traced
    return jnp.where(keep_hi, hi, lo)


def _cmp_ex_const(x, K, j):
    n, C = x.shape
    if j < _ROLL_MAX_J:
        return _stage_roll(x, j, K, None)
    nb = n // (2 * j)
    a = x.reshape(nb, 2, j, C)
    lo = jnp.minimum(a[:, 0], a[:, 1])
    hi = jnp.maximum(a[:, 0], a[:, 1])
    if (nb - 1) * 2 * j < K:
        first, second = lo, hi
    else:
        shift = (K // (2 * j)).bit_length() - 1
        b = lax.broadcasted_iota(jnp.int32, (nb, 1, 1), 0)
        m = (b >> shift) % 2 == 1
        first = jnp.where(m, hi, lo)
        second = jnp.where(m, lo, hi)
    return jnp.stack([first, second], axis=1).reshape(n, C)


def _cmp_ex_dir(x, j, desc):
    n, C = x.shape
    if j < _ROLL_MAX_J:
        return _stage_roll(x, j, None, desc)
    nb = n // (2 * j)
    a = x.reshape(nb, 2, j, C)
    lo = jnp.minimum(a[:, 0], a[:, 1])
    hi = jnp.maximum(a[:, 0], a[:, 1])
    first = jnp.where(desc, hi, lo)
    second = jnp.where(desc, lo, hi)
    return jnp.stack([first, second], axis=1).reshape(n, C)


def _merge_dir(x, desc):
    j = x.shape[0] // 2
    while j >= 1:
        x = _cmp_ex_dir(x, j, desc)
        j //= 2
    return x


def _local_sort(x, parity):
    n = x.shape[0]
    K = 2
    while K <= n // 2:
        j = K // 2
        while j >= 1:
            x = _cmp_ex_const(x, K, j)
            j //= 2
        K *= 2
    return _merge_dir(x, parity)


def _merge_asc(x):
    j = x.shape[0] // 2
    while j >= 1:
        x = _cmp_ex_const(x, 2 * x.shape[0], j)
        j //= 2
    return x


def kernel(x):
    m_per, n_cols = x.shape
    half = n_cols // 2

    def body(x_ref, out_ref, work_a, work_b, comm_ref, send_sems, recv_sems):
        my = lax.axis_index("i")
        p1 = my ^ 1
        p2 = my ^ 2

        barrier_sem = pltpu.get_barrier_semaphore()
        for nbr in (p1, p2):
            pl.semaphore_signal(
                barrier_sem, inc=1,
                device_id=(nbr,), device_id_type=pl.DeviceIdType.MESH,
            )
        pl.semaphore_wait(barrier_sem, 2)

        def start_ex(slot, partner, work_ref, val):
            work_ref[...] = val
            rdma = pltpu.make_async_remote_copy(
                src_ref=work_ref,
                dst_ref=comm_ref.at[slot],
                send_sem=send_sems.at[slot],
                recv_sem=recv_sems.at[slot],
                device_id=(partner,),
                device_id_type=pl.DeviceIdType.MESH,
            )
            rdma.start()
            return rdma

        def finish_ex(rdma, slot, keep_max, val):
            rdma.wait()
            r = comm_ref[slot]
            return jnp.where(keep_max, jnp.maximum(val, r), jnp.minimum(val, r))

        parity = my % 2 == 1
        upper = my >= 2
        km0 = (my == 1) | (my == 2)

        xa = x_ref[:, :half].astype(jnp.bfloat16)
        xb = x_ref[:, half:].astype(jnp.bfloat16)

        xa = _local_sort(xa, parity)
        ra0 = start_ex(0, p1, work_a, xa)
        xb = _local_sort(xb, parity)
        rb0 = start_ex(1, p1, work_b, xb)

        xa = finish_ex(ra0, 0, km0, xa)
        xa = _merge_dir(xa, upper)
        ra1 = start_ex(2, p2, work_a, xa)
        xb = finish_ex(rb0, 1, km0, xb)
        xb = _merge_dir(xb, upper)
        rb1 = start_ex(3, p2, work_b, xb)

        xa = finish_ex(ra1, 2, upper, xa)
        ra2 = start_ex(4, p1, work_a, xa)
        xb = finish_ex(rb1, 3, upper, xb)
        rb2 = start_ex(5, p1, work_b, xb)

        xa = finish_ex(ra2, 4, parity, xa)
        xa = _merge_asc(xa)
        out_ref[:, :half] = xa.astype(jnp.float32)
        xb = finish_ex(rb2, 5, parity, xb)
        xb = _merge_asc(xb)
        out_ref[:, half:] = xb.astype(jnp.float32)

    return pl.pallas_call(
        body,
        out_shape=jax.ShapeDtypeStruct((m_per, n_cols), jnp.float32),
        in_specs=[pl.BlockSpec(memory_space=pltpu.VMEM)],
        out_specs=pl.BlockSpec(memory_space=pltpu.VMEM),
        scratch_shapes=[
            pltpu.VMEM((m_per, half), jnp.bfloat16),
            pltpu.VMEM((m_per, half), jnp.bfloat16),
            pltpu.VMEM((6, m_per, half), jnp.bfloat16),
            pltpu.SemaphoreType.DMA((6,)),
            pltpu.SemaphoreType.DMA((6,)),
        ],
        compiler_params=pltpu.CompilerParams(
            collective_id=0, vmem_limit_bytes=100 * 1024 * 1024
        ),
    )(x)


# device time: 89888 ns/iter; 2.1200x vs baseline; 1.6376x over previous
import numpy as np

import jax
import jax.numpy as jnp
from jax import lax
from jax.experimental import pallas as pl
from jax.experimental.pallas import tpu as pltpu

N_DEV = 4


import os

_ROLL_MAX_J = int(os.environ.get("SORT_ROLL_MAX_J", "8"))
_NO_COMM = os.environ.get("SORT_NO_COMM", "0") == "1"


def _stage_roll(x, j, K, desc_traced):
    n, C = x.shape
    i = lax.broadcasted_iota(jnp.int32, (n, 1), 0)
    bitj = (i >> (j.bit_length() - 1)) % 2 == 1
    up = jnp.roll(x, -j, axis=0)
    dn = jnp.roll(x, j, axis=0)
    partner = jnp.where(bitj, dn, up)
    lo = jnp.minimum(x, partner)
    hi = jnp.maximum(x, partner)
    keep_hi = bitj
    if K is not None and K < n:
        keep_hi = keep_hi ^ ((i >> (K.bit_length() - 1)) % 2 == 1)
    if desc_traced is not None:
        keep_hi = keep_hi ^ desc_traced
    return jnp.where(keep_hi, hi, lo)


def _cmp_ex_const(x, K, j):
    n, C = x.shape
    if j < _ROLL_MAX_J:
        return _stage_roll(x, j, K, None)
    nb = n // (2 * j)
    a = x.reshape(nb, 2, j, C)
    lo = jnp.minimum(a[:, 0], a[:, 1])
    hi = jnp.maximum(a[:, 0], a[:, 1])
    if (nb - 1) * 2 * j < K:
        first, second = lo, hi
    else:
        shift = (K // (2 * j)).bit_length() - 1
        b = lax.broadcasted_iota(jnp.int32, (nb, 1, 1), 0)
        m = (b >> shift) % 2 == 1
        first = jnp.where(m, hi, lo)
        second = jnp.where(m, lo, hi)
    return jnp.stack([first, second], axis=1).reshape(n, C)


def _cmp_ex_dir(x, j, desc):
    n, C = x.shape
    if j < _ROLL_MAX_J:
        return _stage_roll(x, j, None, desc)
    nb = n // (2 * j)
    a = x.reshape(nb, 2, j, C)
    lo = jnp.minimum(a[:, 0], a[:, 1])
    hi = jnp.maximum(a[:, 0], a[:, 1])
    first = jnp.where(desc, hi, lo)
    second = jnp.where(desc, lo, hi)
    return jnp.stack([first, second], axis=1).reshape(n, C)


def _merge_dir(x, desc):
    j = x.shape[0] // 2
    while j >= 1:
        x = _cmp_ex_dir(x, j, desc)
        j //= 2
    return x


def _local_sort(x, parity):
    n = x.shape[0]
    K = 2
    while K <= n // 2:
        j = K // 2
        while j >= 1:
            x = _cmp_ex_const(x, K, j)
            j //= 2
        K *= 2
    return _merge_dir(x, parity)


def _merge_asc(x):
    j = x.shape[0] // 2
    while j >= 1:
        x = _cmp_ex_const(x, 2 * x.shape[0], j)
        j //= 2
    return x


def kernel(x):
    m_per, n_cols = x.shape
    half = n_cols // 2

    def body(x_ref, out_ref, work_a, work_b, comm_ref, send_sems, recv_sems):
        my = lax.axis_index("i")
        p1 = my ^ 1
        p2 = my ^ 2

        barrier_sem = pltpu.get_barrier_semaphore()
        for nbr in (p1, p2):
            pl.semaphore_signal(
                barrier_sem, inc=1,
                device_id=(nbr,), device_id_type=pl.DeviceIdType.MESH,
            )
        pl.semaphore_wait(barrier_sem, 2)

        def start_ex(slot, partner, work_ref, val):
            if _NO_COMM:
                return None
            work_ref[...] = val
            rdma = pltpu.make_async_remote_copy(
                src_ref=work_ref,
                dst_ref=comm_ref.at[slot],
                send_sem=send_sems.at[slot],
                recv_sem=recv_sems.at[slot],
                device_id=(partner,),
                device_id_type=pl.DeviceIdType.MESH,
            )
            rdma.start()
            return rdma

        def finish_ex(rdma, slot, keep_max, val):
            if _NO_COMM:
                return val
            rdma.wait()
            r = comm_ref[slot]
            return jnp.where(keep_max, jnp.maximum(val, r), jnp.minimum(val, r))

        parity = my % 2 == 1
        upper = my >= 2
        km0 = (my == 1) | (my == 2)

        xa = x_ref[:, :half].astype(jnp.bfloat16)
        xb = x_ref[:, half:].astype(jnp.bfloat16)

        xa = _local_sort(xa, parity)
        ra0 = start_ex(0, p1, work_a, xa)
        xb = _local_sort(xb, parity)
        rb0 = start_ex(1, p1, work_b, xb)

        xa = finish_ex(ra0, 0, km0, xa)
        xa = _merge_dir(xa, upper)
        ra1 = start_ex(2, p2, work_a, xa)
        xb = finish_ex(rb0, 1, km0, xb)
        xb = _merge_dir(xb, upper)
        rb1 = start_ex(3, p2, work_b, xb)

        xa = finish_ex(ra1, 2, upper, xa)
        ra2 = start_ex(4, p1, work_a, xa)
        xb = finish_ex(rb1, 3, upper, xb)
        rb2 = start_ex(5, p1, work_b, xb)

        xa = finish_ex(ra2, 4, parity, xa)
        xa = _merge_asc(xa)
        out_ref[:, :half] = xa.astype(jnp.float32)
        xb = finish_ex(rb2, 5, parity, xb)
        xb = _merge_asc(xb)
        out_ref[:, half:] = xb.astype(jnp.float32)

    return pl.pallas_call(
        body,
        out_shape=jax.ShapeDtypeStruct((m_per, n_cols), jnp.float32),
        in_specs=[pl.BlockSpec(memory_space=pltpu.VMEM)],
        out_specs=pl.BlockSpec(memory_space=pltpu.VMEM),
        scratch_shapes=[
            pltpu.VMEM((m_per, half), jnp.bfloat16),
            pltpu.VMEM((m_per, half), jnp.bfloat16),
            pltpu.VMEM((6, m_per, half), jnp.bfloat16),
            pltpu.SemaphoreType.DMA((6,)),
            pltpu.SemaphoreType.DMA((6,)),
        ],
        compiler_params=pltpu.CompilerParams(
            collective_id=0, vmem_limit_bytes=100 * 1024 * 1024
        ),
    )(x)
